# Initial kernel scaffold; baseline (speedup 1.0000x reference)
import jax
import jax.numpy as jnp
from jax.experimental import pallas as pl


def kernel(node_tokens, edge_tokens, edge_index, emb_table, W_gnn, W_edge, W_t, b_t, W_lm, b_lm):
    raise NotImplementedError("write your pallas kernel here")


# speedup vs baseline: 2.1825x; 2.1825x over previous
import functools

import jax
import jax.numpy as jnp
from jax import lax
from jax.experimental import pallas as pl
from jax.experimental.pallas import tpu as pltpu
from jax.experimental.pallas import tpu_sc as plsc

VOCAB = 10000
HID = 64
N_NODES = 5000
N_EDGES = 4000
L_NODE = 128
L_EDGE = 16

NC, NS = 2, 16
NW = NC * NS
EP = 4096
SINK = N_NODES

_N_GROUPS = (N_NODES * L_NODE) // (8 * 128)
_E_GROUPS = (EP * L_EDGE) // (8 * 128)

@functools.cache
def _mesh():
    return plsc.VectorSubcoreMesh(core_axis_name="c", subcore_axis_name="s",
                                  num_cores=NC, num_subcores=NS)


def _gather_body(ntok, etok, emb, ne3, ee3, idx8, row8, sem):
    c = lax.axis_index("c")
    s = lax.axis_index("s")
    w = s * NC + c

    def one_group(g, tok_ref, out_ref):
        pltpu.sync_copy(tok_ref.at[pl.ds(8 * g, 8)], idx8)
        cps = [pltpu.async_copy(emb.at[idx8.at[b]], row8.at[b], sem)
               for b in range(8)]
        for cp in cps:
            cp.wait()
        pltpu.sync_copy(row8, out_ref.at[pl.ds(8 * g, 8)])

    n_j = jnp.where(w < (_N_GROUPS % NW), _N_GROUPS // NW + 1, _N_GROUPS // NW)

    def body(j, carry):
        one_group(w + NW * j, ntok, ne3)
        return carry

    lax.fori_loop(0, n_j, body, 0)

    for j in range(_E_GROUPS // NW):
        one_group(w + NW * j, etok, ee3)


def _sc_gather(node_tokens, etok2, emb_table):
    f = pl.kernel(
        _gather_body,
        out_type=(
            jax.ShapeDtypeStruct((N_NODES * L_NODE // 128, 128, HID), jnp.float32),
            jax.ShapeDtypeStruct((EP * L_EDGE // 128, 128, HID), jnp.float32),
        ),
        mesh=_mesh(),
        compiler_params=pltpu.CompilerParams(use_tc_tiling_on_sc=False),
        scratch_types=[
            pltpu.VMEM((8, 128), jnp.int32),
            pltpu.VMEM((8, 128, HID), jnp.float32),
            pltpu.SemaphoreType.DMA,
        ],
    )
    return f(node_tokens, etok2, emb_table)


def _mm_acc_kernel(x_ref, w_ref, o_ref):
    @pl.when(pl.program_id(1) == 0)
    def _():
        o_ref[...] = jnp.zeros_like(o_ref)

    o_ref[...] += jnp.dot(x_ref[...], w_ref[...],
                          preferred_element_type=jnp.float32)


def _matmul(x, w, bm, bk):
    m, k = x.shape
    n = w.shape[1]
    return pl.pallas_call(
        _mm_acc_kernel,
        grid=(m // bm, k // bk),
        in_specs=[pl.BlockSpec((bm, bk), lambda i, j: (i, j)),
                  pl.BlockSpec((bk, n), lambda i, j: (j, 0))],
        out_specs=pl.BlockSpec((bm, n), lambda i, j: (i, 0)),
        out_shape=jax.ShapeDtypeStruct((m, n), jnp.float32),
        compiler_params=pltpu.CompilerParams(
            dimension_semantics=("parallel", "arbitrary")),
    )(x, w)


def _agg_body(h0, msg_e, ei, agg_out, agg_sh, zbuf, sidx, didx, vb1, vb2, sem):
    c = lax.axis_index("c")
    t = lax.axis_index("s")

    def zrow(i, carry):
        for j in range(HID // 16):
            zbuf[i, pl.ds(j * 16, 16)] = jnp.zeros((16,), jnp.float32)
        return carry

    lax.fori_loop(0, 320, zrow, 0)

    @pl.when(t < 15)
    def _():
        pltpu.sync_copy(zbuf, agg_sh.at[pl.ds(t * 320, 320)])

    @pl.when(t == 15)
    def _():
        pltpu.sync_copy(zbuf.at[pl.ds(0, 208)], agg_sh.at[pl.ds(4800, 208)])

    plsc.subcore_barrier()

    e0 = c * (EP // NC) + t * 128
    pltpu.sync_copy(ei.at[0, pl.ds(e0, 128)], sidx)
    pltpu.sync_copy(ei.at[1, pl.ds(e0, 128)], didx)
    pltpu.sync_copy(msg_e.at[pl.ds(e0, 128)], vb1)
    pltpu.sync_copy(vb1, agg_sh.at[didx], add=True)
    pltpu.async_copy(h0.at[sidx], vb2, sem).wait()
    pltpu.sync_copy(vb2, agg_sh.at[didx], add=True)

    plsc.subcore_barrier()

    @pl.when(t < 15)
    def _():
        pltpu.sync_copy(agg_sh.at[pl.ds(t * 320, 320)], zbuf)
        pltpu.sync_copy(zbuf, agg_out.at[c, pl.ds(t * 320, 320)])

    @pl.when(t == 15)
    def _():
        pltpu.sync_copy(agg_sh.at[pl.ds(4800, 208)], zbuf.at[pl.ds(0, 208)])
        pltpu.sync_copy(zbuf.at[pl.ds(0, 208)], agg_out.at[c, pl.ds(4800, 208)])


def _sc_agg(h0, msg_e, ei_pad):
    f = pl.kernel(
        _agg_body,
        out_type=jax.ShapeDtypeStruct((NC, N_NODES + 8, HID), jnp.float32),
        mesh=_mesh(),
        compiler_params=pltpu.CompilerParams(use_tc_tiling_on_sc=False),
        scratch_types=[
            pltpu.VMEM_SHARED((N_NODES + 8, HID), jnp.float32),
            pltpu.VMEM((320, HID), jnp.float32),
            pltpu.VMEM((128,), jnp.int32),
            pltpu.VMEM((128,), jnp.int32),
            pltpu.VMEM((128, HID), jnp.float32),
            pltpu.VMEM((128, HID), jnp.float32),
            pltpu.SemaphoreType.DMA,
        ],
    )
    return f(h0, msg_e, ei_pad)


def _relu_labels_kernel(h0_ref, agg_ref, et_ref, h_ref, lab_ref):
    a = agg_ref[...]
    h_ref[...] = jnp.maximum(
        h0_ref[...] + a[0, :N_NODES, :] + a[1, :N_NODES, :], 0.0)
    tok = et_ref[...]
    special = tok <= 3
    pseudo = (tok * 131071) % 100 < 15
    masked = pseudo & (~special)
    lab_ref[...] = jnp.where(masked, tok, -100)


def _tc_relu_labels(h0, aggs, edge_tokens):
    return pl.pallas_call(
        _relu_labels_kernel,
        out_shape=(jax.ShapeDtypeStruct((N_NODES, HID), jnp.float32),
                   jax.ShapeDtypeStruct((N_EDGES, L_EDGE), jnp.int32)),
    )(h0, aggs, edge_tokens)


def _epgather_body(h, ei, gs, gd, idxb, vb, sem):
    c = lax.axis_index("c")
    s = lax.axis_index("s")
    w = s * NC + c
    e0 = w * 128
    for d, out in ((0, gs), (1, gd)):
        pltpu.sync_copy(ei.at[d, pl.ds(e0, 128)], idxb)
        pltpu.async_copy(h.at[idxb], vb, sem).wait()
        pltpu.sync_copy(vb, out.at[pl.ds(e0, 128)])


def _sc_epgather(h, ei_pad):
    f = pl.kernel(
        _epgather_body,
        out_type=(jax.ShapeDtypeStruct((EP, HID), jnp.float32),
                  jax.ShapeDtypeStruct((EP, HID), jnp.float32)),
        mesh=_mesh(),
        compiler_params=pltpu.CompilerParams(use_tc_tiling_on_sc=False),
        scratch_types=[
            pltpu.VMEM((128,), jnp.int32),
            pltpu.VMEM((128, HID), jnp.float32),
            pltpu.SemaphoreType.DMA,
        ],
    )
    return f(h, ei_pad)


def _head_kernel(gs_ref, gd_ref, wt_ref, bt_ref, wl_ref, bl_ref, o_ref):
    e = gs_ref[...] + gd_ref[...]
    s = jnp.tanh(jnp.dot(e, wt_ref[...],
                         preferred_element_type=jnp.float32) + bt_ref[...])
    logits = jnp.dot(s, wl_ref[...],
                     preferred_element_type=jnp.float32) + bl_ref[...]
    m = jnp.max(logits, axis=-1, keepdims=True)
    ex = jnp.exp(logits - m)
    o_ref[...] = ex / jnp.sum(ex, axis=-1, keepdims=True)


def _tc_head(g_src, g_dst, W_t, b_t, W_lm, b_lm):
    bm = 400
    return pl.pallas_call(
        _head_kernel,
        grid=(N_EDGES // bm,),
        in_specs=[pl.BlockSpec((bm, HID), lambda i: (i, 0)),
                  pl.BlockSpec((bm, HID), lambda i: (i, 0)),
                  pl.BlockSpec((HID, HID), lambda i: (0, 0)),
                  pl.BlockSpec((1, HID), lambda i: (0, 0)),
                  pl.BlockSpec((HID, VOCAB), lambda i: (0, 0)),
                  pl.BlockSpec((1, VOCAB), lambda i: (0, 0))],
        out_specs=pl.BlockSpec((bm, VOCAB), lambda i: (i, 0)),
        out_shape=jax.ShapeDtypeStruct((N_EDGES, VOCAB), jnp.float32),
    )(g_src, g_dst, W_t, b_t, W_lm, b_lm)


def kernel(node_tokens, edge_tokens, edge_index, emb_table,
           W_gnn, W_edge, W_t, b_t, W_lm, b_lm):
    i32 = jnp.int32
    npad = EP - N_EDGES
    etok_pad = jnp.concatenate(
        [edge_tokens, jnp.zeros((npad, L_EDGE), i32)], axis=0)
    etok2 = etok_pad.reshape(EP * L_EDGE // 128, 128)
    ei_pad = jnp.concatenate(
        [edge_index,
         jnp.stack([jnp.zeros((npad,), i32),
                    jnp.full((npad,), SINK, i32)])], axis=1)

    ne3, ee3 = _sc_gather(node_tokens, etok2, emb_table)
    node_emb = ne3.reshape(N_NODES, L_NODE * HID)
    edge_emb = ee3.reshape(EP, L_EDGE * HID)

    h0 = _matmul(node_emb, W_gnn, bm=1000, bk=1024)
    msg_e = _matmul(edge_emb, W_edge, bm=2048, bk=1024)

    aggs = _sc_agg(h0, msg_e, ei_pad)
    h, labels = _tc_relu_labels(h0, aggs, edge_tokens)
    g_src, g_dst = _sc_epgather(h, ei_pad)

    probs = _tc_head(g_src, g_dst, W_t, b_t.reshape(1, HID),
                     W_lm, b_lm.reshape(1, VOCAB))
    return (labels, probs.reshape(N_EDGES, 1, VOCAB))


# speedup vs baseline: 3.8497x; 1.7638x over previous
import functools

import jax
import jax.numpy as jnp
from jax import lax
from jax.experimental import pallas as pl
from jax.experimental.pallas import tpu as pltpu
from jax.experimental.pallas import tpu_sc as plsc

VOCAB = 10000
HID = 64
N_NODES = 5000
N_EDGES = 4000
L_NODE = 128
L_EDGE = 16

NC, NS = 2, 16
NW = NC * NS
EP = 4096
SINK = N_NODES

_N_GROUPS = (N_NODES * L_NODE) // (8 * 128)
_E_GROUPS = (EP * L_EDGE) // (8 * 128)

@functools.cache
def _mesh():
    return plsc.VectorSubcoreMesh(core_axis_name="c", subcore_axis_name="s",
                                  num_cores=NC, num_subcores=NS)


def _gather_body(ntok, etok, emb, ne3, ee3, idx8, row8, sem):
    c = lax.axis_index("c")
    s = lax.axis_index("s")
    w = s * NC + c

    def one_group(g, tok_ref, out_ref):
        pltpu.sync_copy(tok_ref.at[pl.ds(8 * g, 8)], idx8)
        cps = [pltpu.async_copy(emb.at[idx8.at[b]], row8.at[b], sem)
               for b in range(8)]
        for cp in cps:
            cp.wait()
        pltpu.sync_copy(row8, out_ref.at[pl.ds(8 * g, 8)])

    n_j = jnp.where(w < (_N_GROUPS % NW), _N_GROUPS // NW + 1, _N_GROUPS // NW)

    def body(j, carry):
        one_group(w + NW * j, ntok, ne3)
        return carry

    lax.fori_loop(0, n_j, body, 0)

    for j in range(_E_GROUPS // NW):
        one_group(w + NW * j, etok, ee3)


def _sc_gather(node_tokens, etok2, emb_table):
    f = pl.kernel(
        _gather_body,
        out_type=(
            jax.ShapeDtypeStruct((N_NODES * L_NODE // 128, 128, HID), jnp.bfloat16),
            jax.ShapeDtypeStruct((EP * L_EDGE // 128, 128, HID), jnp.bfloat16),
        ),
        mesh=_mesh(),
        compiler_params=pltpu.CompilerParams(use_tc_tiling_on_sc=False),
        scratch_types=[
            pltpu.VMEM((8, 128), jnp.int32),
            pltpu.VMEM((8, 128, HID), jnp.bfloat16),
            pltpu.SemaphoreType.DMA,
        ],
    )
    return f(node_tokens, etok2, emb_table)


def _mm_acc_kernel(x_ref, w_ref, o_ref):
    @pl.when(pl.program_id(1) == 0)
    def _():
        o_ref[...] = jnp.zeros_like(o_ref)

    o_ref[...] += jnp.dot(x_ref[...], w_ref[...].astype(jnp.bfloat16),
                          preferred_element_type=jnp.float32)


def _matmul(x, w, bm, bk):
    m, k = x.shape
    n = w.shape[1]
    return pl.pallas_call(
        _mm_acc_kernel,
        grid=(m // bm, k // bk),
        in_specs=[pl.BlockSpec((bm, bk), lambda i, j: (i, j)),
                  pl.BlockSpec((bk, n), lambda i, j: (j, 0))],
        out_specs=pl.BlockSpec((bm, n), lambda i, j: (i, 0)),
        out_shape=jax.ShapeDtypeStruct((m, n), jnp.float32),
        compiler_params=pltpu.CompilerParams(
            dimension_semantics=("parallel", "arbitrary")),
    )(x, w)


def _agg_body(h0, msg_e, ei, agg_out, agg_sh, zbuf, sidx, didx, vb1, vb2, sem):
    c = lax.axis_index("c")
    t = lax.axis_index("s")

    def zrow(i, carry):
        for j in range(HID // 16):
            zbuf[i, pl.ds(j * 16, 16)] = jnp.zeros((16,), jnp.float32)
        return carry

    lax.fori_loop(0, 320, zrow, 0)

    @pl.when(t < 15)
    def _():
        pltpu.sync_copy(zbuf, agg_sh.at[pl.ds(t * 320, 320)])

    @pl.when(t == 15)
    def _():
        pltpu.sync_copy(zbuf.at[pl.ds(0, 208)], agg_sh.at[pl.ds(4800, 208)])

    plsc.subcore_barrier()

    e0 = c * (EP // NC) + t * 128
    pltpu.sync_copy(ei.at[0, pl.ds(e0, 128)], sidx)
    pltpu.sync_copy(ei.at[1, pl.ds(e0, 128)], didx)
    pltpu.sync_copy(msg_e.at[pl.ds(e0, 128)], vb1)
    pltpu.sync_copy(vb1, agg_sh.at[didx], add=True)
    pltpu.async_copy(h0.at[sidx], vb2, sem).wait()
    pltpu.sync_copy(vb2, agg_sh.at[didx], add=True)

    plsc.subcore_barrier()

    @pl.when(t < 15)
    def _():
        pltpu.sync_copy(agg_sh.at[pl.ds(t * 320, 320)], zbuf)
        pltpu.sync_copy(zbuf, agg_out.at[c, pl.ds(t * 320, 320)])

    @pl.when(t == 15)
    def _():
        pltpu.sync_copy(agg_sh.at[pl.ds(4800, 208)], zbuf.at[pl.ds(0, 208)])
        pltpu.sync_copy(zbuf.at[pl.ds(0, 208)], agg_out.at[c, pl.ds(4800, 208)])


def _sc_agg(h0, msg_e, ei_pad):
    f = pl.kernel(
        _agg_body,
        out_type=jax.ShapeDtypeStruct((NC, N_NODES + 8, HID), jnp.float32),
        mesh=_mesh(),
        compiler_params=pltpu.CompilerParams(use_tc_tiling_on_sc=False),
        scratch_types=[
            pltpu.VMEM_SHARED((N_NODES + 8, HID), jnp.float32),
            pltpu.VMEM((320, HID), jnp.float32),
            pltpu.VMEM((128,), jnp.int32),
            pltpu.VMEM((128,), jnp.int32),
            pltpu.VMEM((128, HID), jnp.float32),
            pltpu.VMEM((128, HID), jnp.float32),
            pltpu.SemaphoreType.DMA,
        ],
    )
    return f(h0, msg_e, ei_pad)


def _relu_labels_kernel(h0_ref, agg_ref, et_ref, h_ref, lab_ref):
    a = agg_ref[...]
    h_ref[...] = jnp.maximum(
        h0_ref[...] + a[0, :N_NODES, :] + a[1, :N_NODES, :], 0.0)
    tok = et_ref[...]
    special = tok <= 3
    pseudo = (tok * 131071) % 100 < 15
    masked = pseudo & (~special)
    lab_ref[...] = jnp.where(masked, tok, -100)


def _tc_relu_labels(h0, aggs, edge_tokens):
    return pl.pallas_call(
        _relu_labels_kernel,
        out_shape=(jax.ShapeDtypeStruct((N_NODES, HID), jnp.float32),
                   jax.ShapeDtypeStruct((N_EDGES, L_EDGE), jnp.int32)),
    )(h0, aggs, edge_tokens)


def _epgather_body(h, ei, gs, gd, idxb, vb, sem):
    c = lax.axis_index("c")
    s = lax.axis_index("s")
    w = s * NC + c
    e0 = w * 128
    for d, out in ((0, gs), (1, gd)):
        pltpu.sync_copy(ei.at[d, pl.ds(e0, 128)], idxb)
        pltpu.async_copy(h.at[idxb], vb, sem).wait()
        pltpu.sync_copy(vb, out.at[pl.ds(e0, 128)])


def _sc_epgather(h, ei_pad):
    f = pl.kernel(
        _epgather_body,
        out_type=(jax.ShapeDtypeStruct((EP, HID), jnp.float32),
                  jax.ShapeDtypeStruct((EP, HID), jnp.float32)),
        mesh=_mesh(),
        compiler_params=pltpu.CompilerParams(use_tc_tiling_on_sc=False),
        scratch_types=[
            pltpu.VMEM((128,), jnp.int32),
            pltpu.VMEM((128, HID), jnp.float32),
            pltpu.SemaphoreType.DMA,
        ],
    )
    return f(h, ei_pad)


def _head_kernel(gs_ref, gd_ref, wt_ref, bt_ref, wl_ref, bl_ref, o_ref):
    e = gs_ref[...] + gd_ref[...]
    s = jnp.tanh(jnp.dot(e, wt_ref[...],
                         preferred_element_type=jnp.float32) + bt_ref[...])
    logits = jnp.dot(s, wl_ref[...],
                     preferred_element_type=jnp.float32) + bl_ref[...]
    m = jnp.max(logits, axis=-1, keepdims=True)
    ex = jnp.exp(logits - m)
    o_ref[...] = (ex / jnp.sum(ex, axis=-1, keepdims=True))[:, None, :]


def _tc_head(g_src, g_dst, W_t, b_t, W_lm, b_lm):
    bm = 400
    return pl.pallas_call(
        _head_kernel,
        grid=(N_EDGES // bm,),
        in_specs=[pl.BlockSpec((bm, HID), lambda i: (i, 0)),
                  pl.BlockSpec((bm, HID), lambda i: (i, 0)),
                  pl.BlockSpec((HID, HID), lambda i: (0, 0)),
                  pl.BlockSpec((1, HID), lambda i: (0, 0)),
                  pl.BlockSpec((HID, VOCAB), lambda i: (0, 0)),
                  pl.BlockSpec((1, VOCAB), lambda i: (0, 0))],
        out_specs=pl.BlockSpec((bm, 1, VOCAB), lambda i: (i, 0, 0)),
        out_shape=jax.ShapeDtypeStruct((N_EDGES, 1, VOCAB), jnp.float32),
    )(g_src, g_dst, W_t, b_t, W_lm, b_lm)


def kernel(node_tokens, edge_tokens, edge_index, emb_table,
           W_gnn, W_edge, W_t, b_t, W_lm, b_lm):
    i32 = jnp.int32
    npad = EP - N_EDGES
    etok_pad = jnp.concatenate(
        [edge_tokens, jnp.zeros((npad, L_EDGE), i32)], axis=0)
    etok2 = etok_pad.reshape(EP * L_EDGE // 128, 128)
    ei_pad = jnp.concatenate(
        [edge_index,
         jnp.stack([jnp.zeros((npad,), i32),
                    jnp.full((npad,), SINK, i32)])], axis=1)

    ne3, ee3 = _sc_gather(node_tokens, etok2,
                          emb_table.astype(jnp.bfloat16))
    node_emb = ne3.reshape(N_NODES, L_NODE * HID)
    edge_emb = ee3.reshape(EP, L_EDGE * HID)

    h0 = _matmul(node_emb, W_gnn, bm=1000, bk=1024)
    msg_e = _matmul(edge_emb, W_edge, bm=2048, bk=1024)

    aggs = _sc_agg(h0, msg_e, ei_pad)
    h, labels = _tc_relu_labels(h0, aggs, edge_tokens)
    g_src, g_dst = _sc_epgather(h, ei_pad)

    probs = _tc_head(g_src, g_dst, W_t, b_t.reshape(1, HID),
                     W_lm, b_lm.reshape(1, VOCAB))
    return (labels, probs)


# speedup vs baseline: 7.4929x; 1.9464x over previous
import functools

import jax
import jax.numpy as jnp
from jax import lax
from jax.experimental import pallas as pl
from jax.experimental.pallas import tpu as pltpu
from jax.experimental.pallas import tpu_sc as plsc

VOCAB = 10000
HID = 64
N_NODES = 5000
N_EDGES = 4000
L_NODE = 128
L_EDGE = 16

NC, NS = 2, 16
NW = NC * NS
EP = 4096
SINK = N_NODES

_N_GROUPS = (N_NODES * L_NODE) // (8 * 128)
_E_GROUPS = (EP * L_EDGE) // (8 * 128)

@functools.cache
def _mesh():
    return plsc.VectorSubcoreMesh(core_axis_name="c", subcore_axis_name="s",
                                  num_cores=NC, num_subcores=NS)


def _gather_body(ntok, etok, emb, ne3, ee3, idx8, row8, sem):
    c = lax.axis_index("c")
    s = lax.axis_index("s")
    w = s * NC + c

    def one_group(g, tok_ref, out_ref):
        pltpu.sync_copy(tok_ref.at[pl.ds(8 * g, 8)], idx8)
        cps = [pltpu.async_copy(emb.at[idx8.at[b]], row8.at[b], sem)
               for b in range(8)]
        for cp in cps:
            cp.wait()
        pltpu.sync_copy(row8, out_ref.at[pl.ds(8 * g, 8)])

    n_j = jnp.where(w < (_N_GROUPS % NW), _N_GROUPS // NW + 1, _N_GROUPS // NW)

    def body(j, carry):
        one_group(w + NW * j, ntok, ne3)
        return carry

    lax.fori_loop(0, n_j, body, 0)

    for j in range(_E_GROUPS // NW):
        one_group(w + NW * j, etok, ee3)


def _sc_gather(node_tokens, etok2, emb_table):
    f = pl.kernel(
        _gather_body,
        out_type=(
            jax.ShapeDtypeStruct((N_NODES * L_NODE // 128, 128, HID), jnp.float32),
            jax.ShapeDtypeStruct((EP * L_EDGE // 128, 128, HID), jnp.float32),
        ),
        mesh=_mesh(),
        compiler_params=pltpu.CompilerParams(use_tc_tiling_on_sc=False),
        scratch_types=[
            pltpu.VMEM((8, 128), jnp.int32),
            pltpu.VMEM((8, 128, HID), jnp.float32),
            pltpu.SemaphoreType.DMA,
        ],
    )
    return f(node_tokens, etok2, emb_table)


def _paired_mm_kernel(rpr, x_ref, w_ref, o_ref):
    bm = o_ref.shape[0]
    x = x_ref[...].astype(jnp.bfloat16).reshape(bm, rpr, 128)
    nj = w_ref.shape[0]
    acc = None
    for j in range(nj):
        xj = jnp.concatenate([x[:, 4 * j + i, :] for i in range(4)], axis=-1)
        d = jnp.dot(xj, w_ref[j].astype(jnp.bfloat16),
                    preferred_element_type=jnp.float32)
        acc = d if acc is None else acc + d
    o_ref[...] = acc


def _paired_matmul(x2, w, bm):
    rows, _ = x2.shape
    k, n = w.shape
    rpr = k // 128
    m = rows // rpr
    w4 = w.reshape(k // 512, 512, n)
    return pl.pallas_call(
        functools.partial(_paired_mm_kernel, rpr),
        grid=(m // bm,),
        in_specs=[pl.BlockSpec((bm * rpr, 128), lambda i: (i, 0)),
                  pl.BlockSpec((k // 512, 512, n), lambda i: (0, 0, 0))],
        out_specs=pl.BlockSpec((bm, n), lambda i: (i, 0)),
        out_shape=jax.ShapeDtypeStruct((m, n), jnp.float32),
    )(x2, w4)


def _agg_body(h0, msg_e, ei, agg_out, agg_sh, zbuf, sidx, didx, vb1, vb2, sem):
    c = lax.axis_index("c")
    t = lax.axis_index("s")

    def zrow(i, carry):
        for j in range(HID // 16):
            zbuf[i, pl.ds(j * 16, 16)] = jnp.zeros((16,), jnp.float32)
        return carry

    lax.fori_loop(0, 320, zrow, 0)

    @pl.when(t < 15)
    def _():
        pltpu.sync_copy(zbuf, agg_sh.at[pl.ds(t * 320, 320)])

    @pl.when(t == 15)
    def _():
        pltpu.sync_copy(zbuf.at[pl.ds(0, 208)], agg_sh.at[pl.ds(4800, 208)])

    plsc.subcore_barrier()

    e0 = c * (EP // NC) + t * 128
    pltpu.sync_copy(ei.at[0, pl.ds(e0, 128)], sidx)
    pltpu.sync_copy(ei.at[1, pl.ds(e0, 128)], didx)
    pltpu.sync_copy(msg_e.at[pl.ds(e0, 128)], vb1)
    pltpu.sync_copy(vb1, agg_sh.at[didx], add=True)
    pltpu.async_copy(h0.at[sidx], vb2, sem).wait()
    pltpu.sync_copy(vb2, agg_sh.at[didx], add=True)

    plsc.subcore_barrier()

    @pl.when(t < 15)
    def _():
        pltpu.sync_copy(agg_sh.at[pl.ds(t * 320, 320)], zbuf)
        pltpu.sync_copy(zbuf, agg_out.at[c, pl.ds(t * 320, 320)])

    @pl.when(t == 15)
    def _():
        pltpu.sync_copy(agg_sh.at[pl.ds(4800, 208)], zbuf.at[pl.ds(0, 208)])
        pltpu.sync_copy(zbuf.at[pl.ds(0, 208)], agg_out.at[c, pl.ds(4800, 208)])


def _sc_agg(h0, msg_e, ei_pad):
    f = pl.kernel(
        _agg_body,
        out_type=jax.ShapeDtypeStruct((NC, N_NODES + 8, HID), jnp.float32),
        mesh=_mesh(),
        compiler_params=pltpu.CompilerParams(use_tc_tiling_on_sc=False),
        scratch_types=[
            pltpu.VMEM_SHARED((N_NODES + 8, HID), jnp.float32),
            pltpu.VMEM((320, HID), jnp.float32),
            pltpu.VMEM((128,), jnp.int32),
            pltpu.VMEM((128,), jnp.int32),
            pltpu.VMEM((128, HID), jnp.float32),
            pltpu.VMEM((128, HID), jnp.float32),
            pltpu.SemaphoreType.DMA,
        ],
    )
    return f(h0, msg_e, ei_pad)


def _relu_labels_kernel(h0_ref, agg_ref, et_ref, h_ref, lab_ref):
    a = agg_ref[...]
    h_ref[...] = jnp.maximum(
        h0_ref[...] + a[0, :N_NODES, :] + a[1, :N_NODES, :], 0.0)
    tok = et_ref[...]
    special = tok <= 3
    pseudo = (tok * 131071) % 100 < 15
    masked = pseudo & (~special)
    lab_ref[...] = jnp.where(masked, tok, -100)


def _tc_relu_labels(h0, aggs, edge_tokens):
    return pl.pallas_call(
        _relu_labels_kernel,
        out_shape=(jax.ShapeDtypeStruct((N_NODES, HID), jnp.float32),
                   jax.ShapeDtypeStruct((N_EDGES, L_EDGE), jnp.int32)),
    )(h0, aggs, edge_tokens)


def _epgather_body(h, ei, gs, gd, idxb, vb, sem):
    c = lax.axis_index("c")
    s = lax.axis_index("s")
    w = s * NC + c
    e0 = w * 128
    for d, out in ((0, gs), (1, gd)):
        pltpu.sync_copy(ei.at[d, pl.ds(e0, 128)], idxb)
        pltpu.async_copy(h.at[idxb], vb, sem).wait()
        pltpu.sync_copy(vb, out.at[pl.ds(e0, 128)])


def _sc_epgather(h, ei_pad):
    f = pl.kernel(
        _epgather_body,
        out_type=(jax.ShapeDtypeStruct((EP, HID), jnp.float32),
                  jax.ShapeDtypeStruct((EP, HID), jnp.float32)),
        mesh=_mesh(),
        compiler_params=pltpu.CompilerParams(use_tc_tiling_on_sc=False),
        scratch_types=[
            pltpu.VMEM((128,), jnp.int32),
            pltpu.VMEM((128, HID), jnp.float32),
            pltpu.SemaphoreType.DMA,
        ],
    )
    return f(h, ei_pad)


def _head_kernel(gs_ref, gd_ref, wt_ref, bt_ref, wl_ref, bl_ref, o_ref):
    e = gs_ref[...] + gd_ref[...]
    s = jnp.tanh(jnp.dot(e, wt_ref[...],
                         preferred_element_type=jnp.float32) + bt_ref[...])
    logits = jnp.dot(s, wl_ref[...],
                     preferred_element_type=jnp.float32) + bl_ref[...]
    m = jnp.max(logits, axis=-1, keepdims=True)
    ex = jnp.exp(logits - m)
    o_ref[...] = (ex / jnp.sum(ex, axis=-1, keepdims=True))[:, None, :]


def _tc_head(g_src, g_dst, W_t, b_t, W_lm, b_lm):
    bm = 400
    return pl.pallas_call(
        _head_kernel,
        grid=(N_EDGES // bm,),
        in_specs=[pl.BlockSpec((bm, HID), lambda i: (i, 0)),
                  pl.BlockSpec((bm, HID), lambda i: (i, 0)),
                  pl.BlockSpec((HID, HID), lambda i: (0, 0)),
                  pl.BlockSpec((1, HID), lambda i: (0, 0)),
                  pl.BlockSpec((HID, VOCAB), lambda i: (0, 0)),
                  pl.BlockSpec((1, VOCAB), lambda i: (0, 0))],
        out_specs=pl.BlockSpec((bm, 1, VOCAB), lambda i: (i, 0, 0)),
        out_shape=jax.ShapeDtypeStruct((N_EDGES, 1, VOCAB), jnp.float32),
    )(g_src, g_dst, W_t, b_t, W_lm, b_lm)


def kernel(node_tokens, edge_tokens, edge_index, emb_table,
           W_gnn, W_edge, W_t, b_t, W_lm, b_lm):
    i32 = jnp.int32
    npad = EP - N_EDGES
    etok_pad = jnp.concatenate(
        [edge_tokens, jnp.zeros((npad, L_EDGE), i32)], axis=0)
    etok2 = etok_pad.reshape(EP * L_EDGE // 128, 128)
    ei_pad = jnp.concatenate(
        [edge_index,
         jnp.stack([jnp.zeros((npad,), i32),
                    jnp.full((npad,), SINK, i32)])], axis=1)

    ne3, ee3 = _sc_gather(node_tokens, etok2, emb_table)
    ne2 = ne3.reshape(N_NODES * L_NODE * HID // 128, 128)
    ee2 = ee3.reshape(EP * L_EDGE * HID // 128, 128)

    h0 = _paired_matmul(ne2, W_gnn, bm=200)
    msg_e = _paired_matmul(ee2, W_edge, bm=2048)

    aggs = _sc_agg(h0, msg_e, ei_pad)
    h, labels = _tc_relu_labels(h0, aggs, edge_tokens)
    g_src, g_dst = _sc_epgather(h, ei_pad)

    probs = _tc_head(g_src, g_dst, W_t, b_t.reshape(1, HID),
                     W_lm, b_lm.reshape(1, VOCAB))
    return (labels, probs)


# speedup vs baseline: 9.3469x; 1.2474x over previous
import functools

import jax
import jax.numpy as jnp
from jax import lax
from jax.experimental import pallas as pl
from jax.experimental.pallas import tpu as pltpu
from jax.experimental.pallas import tpu_sc as plsc

VOCAB = 10000
HID = 64
N_NODES = 5000
N_EDGES = 4000
L_NODE = 128
L_EDGE = 16

NC, NS = 2, 16
NW = NC * NS
EP = 4096
SINK = N_NODES

_N_GROUPS = (N_NODES * L_NODE) // (8 * 128)
_E_GROUPS = (EP * L_EDGE) // (8 * 128)

@functools.cache
def _mesh():
    return plsc.VectorSubcoreMesh(core_axis_name="c", subcore_axis_name="s",
                                  num_cores=NC, num_subcores=NS)


_N_GROUPS4 = N_NODES * L_NODE // (4 * 128)
_E_GROUPS4 = EP * L_EDGE // (4 * 128)


def _gather_body(ntok, etok, emb, ne3, ee3, tbl,
                 idxA, idxB, rowA, rowB, semA, semB):
    c = lax.axis_index("c")
    t = lax.axis_index("s")
    w = t * NC + c

    @pl.when(t < 15)
    def _():
        pltpu.sync_copy(emb.at[pl.ds(t * 640, 640)], tbl.at[pl.ds(t * 640, 640)])

    @pl.when(t == 15)
    def _():
        pltpu.sync_copy(emb.at[pl.ds(9600, 400)], tbl.at[pl.ds(9600, 400)])

    plsc.subcore_barrier()

    nw = _N_GROUPS4 // NW + (w < (_N_GROUPS4 % NW))
    nv = nw + _E_GROUPS4 // NW

    def stage(j, idxX, rowX, semX):
        @pl.when(j < nw)
        def _():
            g = w + NW * j
            pltpu.sync_copy(ntok.at[pl.ds(4 * g, 4)], idxX)
            for b in range(4):
                pltpu.async_copy(tbl.at[idxX.at[b]], rowX.at[b], semX)

        @pl.when((j >= nw) & (j < nv))
        def _():
            ge = w + NW * (j - nw)
            pltpu.sync_copy(etok.at[pl.ds(4 * ge, 4)], idxX)
            for b in range(4):
                pltpu.async_copy(tbl.at[idxX.at[b]], rowX.at[b], semX)

    def drain_out(j, idxX, rowX, semX):
        @pl.when(j < nw)
        def _():
            g = w + NW * j
            for b in range(4):
                pltpu.make_async_copy(tbl.at[idxX.at[b]], rowX.at[b], semX).wait()
            pltpu.sync_copy(rowX, ne3.at[pl.ds(4 * g, 4)])

        @pl.when((j >= nw) & (j < nv))
        def _():
            ge = w + NW * (j - nw)
            for b in range(4):
                pltpu.make_async_copy(tbl.at[idxX.at[b]], rowX.at[b], semX).wait()
            pltpu.sync_copy(rowX, ee3.at[pl.ds(4 * ge, 4)])

    A = (idxA, rowA, semA)
    B = (idxB, rowB, semB)
    stage(0, *A)

    def body(i, carry):
        jA = 2 * i
        stage(jA + 1, *B)
        drain_out(jA, *A)
        stage(jA + 2, *A)
        drain_out(jA + 1, *B)
        return carry

    lax.fori_loop(0, 22, body, 0)


def _sc_gather(node_tokens, etok2, emb_table):
    f = pl.kernel(
        _gather_body,
        out_type=(
            jax.ShapeDtypeStruct((N_NODES * L_NODE // 128, 128, HID), jnp.float32),
            jax.ShapeDtypeStruct((EP * L_EDGE // 128, 128, HID), jnp.float32),
        ),
        mesh=_mesh(),
        compiler_params=pltpu.CompilerParams(use_tc_tiling_on_sc=False),
        scratch_types=[
            pltpu.VMEM_SHARED((VOCAB, HID), jnp.float32),
            pltpu.VMEM((4, 128), jnp.int32),
            pltpu.VMEM((4, 128), jnp.int32),
            pltpu.VMEM((4, 128, HID), jnp.float32),
            pltpu.VMEM((4, 128, HID), jnp.float32),
            pltpu.SemaphoreType.DMA,
            pltpu.SemaphoreType.DMA,
        ],
    )
    return f(node_tokens, etok2, emb_table)


def _paired_mm_kernel(rpr, x_ref, w_ref, o_ref):
    bm = o_ref.shape[0]
    x = x_ref[...].astype(jnp.bfloat16).reshape(bm, rpr, 128)
    nj = w_ref.shape[0]
    acc = None
    for j in range(nj):
        xj = jnp.concatenate([x[:, 4 * j + i, :] for i in range(4)], axis=-1)
        d = jnp.dot(xj, w_ref[j].astype(jnp.bfloat16),
                    preferred_element_type=jnp.float32)
        acc = d if acc is None else acc + d
    o_ref[...] = acc


def _paired_matmul(x2, w, bm):
    rows, _ = x2.shape
    k, n = w.shape
    rpr = k // 128
    m = rows // rpr
    w4 = w.reshape(k // 512, 512, n)
    return pl.pallas_call(
        functools.partial(_paired_mm_kernel, rpr),
        grid=(m // bm,),
        in_specs=[pl.BlockSpec((bm * rpr, 128), lambda i: (i, 0)),
                  pl.BlockSpec((k // 512, 512, n), lambda i: (0, 0, 0))],
        out_specs=pl.BlockSpec((bm, n), lambda i: (i, 0)),
        out_shape=jax.ShapeDtypeStruct((m, n), jnp.float32),
    )(x2, w4)


def _agg_body(h0, msg_e, ei, agg_out, agg_sh, zbuf, sidx, didx, vb1, vb2, sem):
    c = lax.axis_index("c")
    t = lax.axis_index("s")

    def zrow(i, carry):
        for j in range(HID // 16):
            zbuf[i, pl.ds(j * 16, 16)] = jnp.zeros((16,), jnp.float32)
        return carry

    lax.fori_loop(0, 320, zrow, 0)

    @pl.when(t < 15)
    def _():
        pltpu.sync_copy(zbuf, agg_sh.at[pl.ds(t * 320, 320)])

    @pl.when(t == 15)
    def _():
        pltpu.sync_copy(zbuf.at[pl.ds(0, 208)], agg_sh.at[pl.ds(4800, 208)])

    plsc.subcore_barrier()

    e0 = c * (EP // NC) + t * 128
    pltpu.sync_copy(ei.at[0, pl.ds(e0, 128)], sidx)
    pltpu.sync_copy(ei.at[1, pl.ds(e0, 128)], didx)
    pltpu.sync_copy(msg_e.at[pl.ds(e0, 128)], vb1)
    pltpu.sync_copy(vb1, agg_sh.at[didx], add=True)
    pltpu.async_copy(h0.at[sidx], vb2, sem).wait()
    pltpu.sync_copy(vb2, agg_sh.at[didx], add=True)

    plsc.subcore_barrier()

    @pl.when(t < 15)
    def _():
        pltpu.sync_copy(agg_sh.at[pl.ds(t * 320, 320)], zbuf)
        pltpu.sync_copy(zbuf, agg_out.at[c, pl.ds(t * 320, 320)])

    @pl.when(t == 15)
    def _():
        pltpu.sync_copy(agg_sh.at[pl.ds(4800, 208)], zbuf.at[pl.ds(0, 208)])
        pltpu.sync_copy(zbuf.at[pl.ds(0, 208)], agg_out.at[c, pl.ds(4800, 208)])


def _sc_agg(h0, msg_e, ei_pad):
    f = pl.kernel(
        _agg_body,
        out_type=jax.ShapeDtypeStruct((NC, N_NODES + 8, HID), jnp.float32),
        mesh=_mesh(),
        compiler_params=pltpu.CompilerParams(use_tc_tiling_on_sc=False),
        scratch_types=[
            pltpu.VMEM_SHARED((N_NODES + 8, HID), jnp.float32),
            pltpu.VMEM((320, HID), jnp.float32),
            pltpu.VMEM((128,), jnp.int32),
            pltpu.VMEM((128,), jnp.int32),
            pltpu.VMEM((128, HID), jnp.float32),
            pltpu.VMEM((128, HID), jnp.float32),
            pltpu.SemaphoreType.DMA,
        ],
    )
    return f(h0, msg_e, ei_pad)


def _relu_labels_kernel(h0_ref, agg_ref, et_ref, h_ref, lab_ref):
    a = agg_ref[...]
    h_ref[...] = jnp.maximum(
        h0_ref[...] + a[0, :N_NODES, :] + a[1, :N_NODES, :], 0.0)
    tok = et_ref[...]
    special = tok <= 3
    pseudo = (tok * 131071) % 100 < 15
    masked = pseudo & (~special)
    lab_ref[...] = jnp.where(masked, tok, -100)


def _tc_relu_labels(h0, aggs, edge_tokens):
    return pl.pallas_call(
        _relu_labels_kernel,
        out_shape=(jax.ShapeDtypeStruct((N_NODES, HID), jnp.float32),
                   jax.ShapeDtypeStruct((N_EDGES, L_EDGE), jnp.int32)),
    )(h0, aggs, edge_tokens)


def _epgather_body(h, ei, gs, gd, idxb, vb, sem):
    c = lax.axis_index("c")
    s = lax.axis_index("s")
    w = s * NC + c
    e0 = w * 128
    for d, out in ((0, gs), (1, gd)):
        pltpu.sync_copy(ei.at[d, pl.ds(e0, 128)], idxb)
        pltpu.async_copy(h.at[idxb], vb, sem).wait()
        pltpu.sync_copy(vb, out.at[pl.ds(e0, 128)])


def _sc_epgather(h, ei_pad):
    f = pl.kernel(
        _epgather_body,
        out_type=(jax.ShapeDtypeStruct((EP, HID), jnp.float32),
                  jax.ShapeDtypeStruct((EP, HID), jnp.float32)),
        mesh=_mesh(),
        compiler_params=pltpu.CompilerParams(use_tc_tiling_on_sc=False),
        scratch_types=[
            pltpu.VMEM((128,), jnp.int32),
            pltpu.VMEM((128, HID), jnp.float32),
            pltpu.SemaphoreType.DMA,
        ],
    )
    return f(h, ei_pad)


def _head_kernel(gs_ref, gd_ref, wt_ref, bt_ref, wl_ref, bl_ref, o_ref):
    e = gs_ref[...] + gd_ref[...]
    s = jnp.tanh(jnp.dot(e, wt_ref[...],
                         preferred_element_type=jnp.float32) + bt_ref[...])
    logits = jnp.dot(s.astype(jnp.bfloat16), wl_ref[...].astype(jnp.bfloat16),
                     preferred_element_type=jnp.float32) + bl_ref[...]
    m = jnp.max(logits, axis=-1, keepdims=True)
    ex = jnp.exp(logits - m)
    o_ref[...] = (ex / jnp.sum(ex, axis=-1, keepdims=True))[:, None, :]


def _tc_head(g_src, g_dst, W_t, b_t, W_lm, b_lm):
    bm = 400
    return pl.pallas_call(
        _head_kernel,
        grid=(N_EDGES // bm,),
        in_specs=[pl.BlockSpec((bm, HID), lambda i: (i, 0)),
                  pl.BlockSpec((bm, HID), lambda i: (i, 0)),
                  pl.BlockSpec((HID, HID), lambda i: (0, 0)),
                  pl.BlockSpec((1, HID), lambda i: (0, 0)),
                  pl.BlockSpec((HID, VOCAB), lambda i: (0, 0)),
                  pl.BlockSpec((1, VOCAB), lambda i: (0, 0))],
        out_specs=pl.BlockSpec((bm, 1, VOCAB), lambda i: (i, 0, 0)),
        out_shape=jax.ShapeDtypeStruct((N_EDGES, 1, VOCAB), jnp.float32),
    )(g_src, g_dst, W_t, b_t, W_lm, b_lm)


def kernel(node_tokens, edge_tokens, edge_index, emb_table,
           W_gnn, W_edge, W_t, b_t, W_lm, b_lm):
    i32 = jnp.int32
    npad = EP - N_EDGES
    etok_pad = jnp.concatenate(
        [edge_tokens, jnp.zeros((npad, L_EDGE), i32)], axis=0)
    etok2 = etok_pad.reshape(EP * L_EDGE // 128, 128)
    ei_pad = jnp.concatenate(
        [edge_index,
         jnp.stack([jnp.zeros((npad,), i32),
                    jnp.full((npad,), SINK, i32)])], axis=1)

    ne3, ee3 = _sc_gather(node_tokens, etok2, emb_table)
    ne2 = ne3.reshape(N_NODES * L_NODE * HID // 128, 128)
    ee2 = ee3.reshape(EP * L_EDGE * HID // 128, 128)

    h0 = _paired_matmul(ne2, W_gnn, bm=200)
    msg_e = _paired_matmul(ee2, W_edge, bm=2048)

    aggs = _sc_agg(h0, msg_e, ei_pad)
    h, labels = _tc_relu_labels(h0, aggs, edge_tokens)
    g_src, g_dst = _sc_epgather(h, ei_pad)

    probs = _tc_head(g_src, g_dst, W_t, b_t.reshape(1, HID),
                     W_lm, b_lm.reshape(1, VOCAB))
    return (labels, probs)
